# baseline (speedup 1.0000x reference)
import functools

import numpy as np
import jax
import jax.numpy as jnp
from jax import lax
from jax.experimental import pallas as pl
from jax.experimental.pallas import tpu as pltpu
from jax.experimental.pallas import tpu_sc as plsc

ENT_TOT = 1000000
REL_TOT = 1000
REL_PAD = 1024
DIM = 64
PK = 2 * DIM
BATCH = 16384

NUM_CORES = 2
NUM_SUBCORES = 16
NUM_WORKERS = NUM_CORES * NUM_SUBCORES
B_PER_W = BATCH // NUM_WORKERS
CHUNK = 128
N_CHUNKS = B_PER_W // CHUNK
GROUPS = CHUNK // 16

PACK_BLK = 5000

_TAKE_DNUMS = lax.GatherDimensionNumbers(
    offset_dims=(), collapsed_slice_dims=(0,), start_index_map=(0,))


def _lane_permute(v, perm):
    return lax.gather(v, perm[:, None], dimension_numbers=_TAKE_DNUMS,
                      slice_sizes=(1,),
                      mode=lax.GatherScatterMode.PROMISE_IN_BOUNDS)


def _lane_sum(v, perms):
    for perm in perms:
        v = v + _lane_permute(v, perm)
    return v


def _rsqrt_nr(x):
    xi = lax.bitcast_convert_type(x, jnp.int32)
    yi = 0x5F3759DF - lax.shift_right_arithmetic(xi, 1)
    y = lax.bitcast_convert_type(yi, jnp.float32)
    for _ in range(2):
        y = y * (1.5 - 0.5 * x * y * y)
    return y


def _transe_sc(batch_h, batch_t, batch_r, ent_p, rel_p):
    mesh = plsc.VectorSubcoreMesh(core_axis_name="c", subcore_axis_name="s")

    @functools.partial(
        pl.kernel,
        mesh=mesh,
        out_type=jax.ShapeDtypeStruct((BATCH,), jnp.float32),
        scratch_types=[
            pltpu.VMEM((N_CHUNKS, CHUNK), jnp.int32),
            pltpu.VMEM((N_CHUNKS, CHUNK), jnp.int32),
            pltpu.VMEM((N_CHUNKS, CHUNK), jnp.int32),
            pltpu.VMEM((N_CHUNKS, CHUNK), jnp.int32),
            pltpu.VMEM((N_CHUNKS, CHUNK), jnp.int32),
            pltpu.VMEM((N_CHUNKS, CHUNK), jnp.int32),
            pltpu.VMEM((2, CHUNK, PK), jnp.float32),
            pltpu.VMEM((2, CHUNK, PK), jnp.float32),
            pltpu.VMEM((2, CHUNK, PK), jnp.float32),
            pltpu.VMEM((B_PER_W,), jnp.float32),
            pltpu.SemaphoreType.DMA,
            pltpu.SemaphoreType.DMA,
        ],
    )
    def k(bh_hbm, bt_hbm, br_hbm, ent_hbm, rel_hbm, out_hbm,
          ri_h, ri_t, ri_r, pi_h, pi_t, pi_r,
          h_buf, t_buf, r_buf, out_v, sem0, sem1):
        wid = lax.axis_index("s") * NUM_CORES + lax.axis_index("c")
        base = wid * B_PER_W
        sems = (sem0, sem1)

        for c in range(N_CHUNKS):
            off = base + c * CHUNK
            pltpu.sync_copy(bh_hbm.at[pl.ds(off, CHUNK)], ri_h.at[c])
            pltpu.sync_copy(bt_hbm.at[pl.ds(off, CHUNK)], ri_t.at[c])
            pltpu.sync_copy(br_hbm.at[pl.ds(off, CHUNK)], ri_r.at[c])
        for raw, packed in ((ri_h, pi_h), (ri_t, pi_t), (ri_r, pi_r)):
            for c in range(N_CHUNKS):
                for v in range(CHUNK // 16):
                    sl = pl.ds(v * 16, 16)
                    packed[c, sl] = lax.shift_right_logical(raw[c, sl], 1)

        iota16 = lax.iota(jnp.int32, 16)
        perms = [lax.bitwise_xor(iota16, jnp.int32(kk)) for kk in (1, 2, 4, 8)]

        def fire(c, b):
            return [
                pltpu.async_copy(ent_hbm.at[pi_h.at[c]], h_buf.at[b], sems[b]),
                pltpu.async_copy(ent_hbm.at[pi_t.at[c]], t_buf.at[b], sems[b]),
                pltpu.async_copy(rel_hbm.at[pi_r.at[c]], r_buf.at[b], sems[b]),
            ]

        def compute(c, b):
            def group_body(g, _):
                sl = pl.ds(c * CHUNK + g * 16, 16)
                del sl
                hvec = ri_h[c, pl.ds(g * 16, 16)]
                tvec = ri_t[c, pl.ds(g * 16, 16)]
                rvec = ri_r[c, pl.ds(g * 16, 16)]
                acc = jnp.zeros((16,), jnp.float32)
                for j in range(16):
                    def _off(x):
                        return pl.multiple_of(lax.bitwise_and(x, 1) * DIM,
                                              DIM)

                    oh = _off(hvec[j])
                    ot = _off(tvec[j])
                    orr = _off(rvec[j])
                    row = g * 16 + j
                    hv = [h_buf[b, row, pl.ds(oh + 16 * q, 16)]
                          for q in range(4)]
                    tv = [t_buf[b, row, pl.ds(ot + 16 * q, 16)]
                          for q in range(4)]
                    rv = [r_buf[b, row, pl.ds(orr + 16 * q, 16)]
                          for q in range(4)]

                    def inv_norm(vs):
                        ssq = ((vs[0] * vs[0] + vs[1] * vs[1])
                               + (vs[2] * vs[2] + vs[3] * vs[3]))
                        return _rsqrt_nr(
                            jnp.maximum(_lane_sum(ssq, perms), 1e-24))

                    ih = inv_norm(hv)
                    it = inv_norm(tv)
                    ir = inv_norm(rv)

                    s = jnp.zeros((16,), jnp.float32)
                    for q in range(4):
                        s = s + jnp.abs(hv[q] * ih + rv[q] * ir - tv[q] * it)
                    score = _lane_sum(s, perms)
                    acc = jnp.where(iota16 == j, score, acc)
                out_v[pl.ds(c * CHUNK + g * 16, 16)] = acc
                return 0

            lax.fori_loop(0, GROUPS, group_body, 0)

        cps0 = fire(0, 0)
        cps1 = fire(1, 1)
        for cp in cps0:
            cp.wait()
        compute(0, 0)
        cps2 = fire(2, 0)
        for cp in cps1:
            cp.wait()
        compute(1, 1)
        cps3 = fire(3, 1)
        for cp in cps2:
            cp.wait()
        compute(2, 0)
        for cp in cps3:
            cp.wait()
        compute(3, 1)

        pltpu.sync_copy(out_v, out_hbm.at[pl.ds(base, B_PER_W)])

    return k(batch_h, batch_t, batch_r, ent_p, rel_p)


PACK_TILES = 20
PACK_ROWS = PACK_TILES * 8
PACK_CHUNKS = ENT_TOT // PACK_ROWS


def _pack_sc(ent_emb):
    mesh = plsc.VectorSubcoreMesh(core_axis_name="c", subcore_axis_name="s")

    @functools.partial(
        pl.kernel,
        mesh=mesh,
        out_type=jax.ShapeDtypeStruct((ENT_TOT // 2, PK), jnp.float32),
        scratch_types=[
            pltpu.VMEM((2, PACK_TILES, 8, DIM), jnp.float32),
            pltpu.VMEM((2, PACK_ROWS // 2, PK), jnp.float32),
            pltpu.SemaphoreType.DMA,
            pltpu.SemaphoreType.DMA,
            pltpu.SemaphoreType.DMA,
            pltpu.SemaphoreType.DMA,
        ],
    )
    def k(ent_hbm, out_hbm, ibuf, obuf, isem0, isem1, osem0, osem1):
        wid = lax.axis_index("s") * NUM_CORES + lax.axis_index("c")
        ent_t = ent_hbm.reshape(ENT_TOT // 8, 8, DIM)
        isems = (isem0, isem1)
        osems = (osem0, osem1)

        def fire_in(c, b):
            return pltpu.async_copy(
                ent_t.at[pl.ds(c * PACK_TILES, PACK_TILES)], ibuf.at[b],
                isems[b])

        def compact(b):
            def tile_body(t, _):
                for s in range(8):
                    pr = t * 4 + (s // 2)
                    off = (s % 2) * DIM
                    for q in range(4):
                        obuf[b, pr, pl.ds(off + 16 * q, 16)] = (
                            ibuf[b, t, s, pl.ds(16 * q, 16)])
                return 0
            lax.fori_loop(0, PACK_TILES, tile_body, 0)

        def fire_out(c, b):
            return pltpu.async_copy(
                obuf.at[b],
                out_hbm.at[pl.ds(c * (PACK_ROWS // 2), PACK_ROWS // 2)],
                osems[b])

        def step_body(g, _):
            c0 = (4 * g) * NUM_WORKERS + wid
            c1 = (4 * g + 1) * NUM_WORKERS + wid
            c2 = (4 * g + 2) * NUM_WORKERS + wid
            c3 = (4 * g + 3) * NUM_WORKERS + wid
            i0 = fire_in(c0, 0)
            i1 = fire_in(c1, 1)
            i0.wait()
            compact(0)
            o0 = fire_out(c0, 0)
            i2 = fire_in(c2, 0)
            i1.wait()
            compact(1)
            o1 = fire_out(c1, 1)
            i3 = fire_in(c3, 1)
            i2.wait()
            o0.wait()
            compact(0)
            o2 = fire_out(c2, 0)
            i3.wait()
            o1.wait()
            compact(1)
            o3 = fire_out(c3, 1)
            o2.wait()
            o3.wait()
            return 0

        lax.fori_loop(0, PACK_CHUNKS // (4 * NUM_WORKERS), step_body, 0)

        tail_start = (PACK_CHUNKS // (4 * NUM_WORKERS)) * 4 * NUM_WORKERS
        n_tail = PACK_CHUNKS - tail_start
        for t in range(-(-n_tail // NUM_WORKERS)):
            c = tail_start + t * NUM_WORKERS + wid

            @pl.when(c < PACK_CHUNKS)
            def _(c=c):
                fire_in(c, 0).wait()
                compact(0)
                fire_out(c, 0).wait()

    return k(ent_emb)


def kernel(batch_h, batch_t, batch_r, ent_emb, rel_emb):
    ent_p = _pack_sc(ent_emb)
    rel_pad = jnp.pad(rel_emb, ((0, REL_PAD - REL_TOT), (0, 0)))
    rel_p = jnp.reshape(rel_pad, (REL_PAD // 2, PK))
    return _transe_sc(batch_h, batch_t, batch_r, ent_p, rel_p)


# speedup vs baseline: 2.2989x; 2.2989x over previous
import functools

import numpy as np
import jax
import jax.numpy as jnp
from jax import lax
from jax.experimental import pallas as pl
from jax.experimental.pallas import tpu as pltpu
from jax.experimental.pallas import tpu_sc as plsc

ENT_TOT = 1000000
REL_TOT = 1000
DIM = 64
SUB = 8
BATCH = 16384

NUM_CORES = 2
NUM_SUBCORES = 16
NUM_WORKERS = NUM_CORES * NUM_SUBCORES
B_PER_W = BATCH // NUM_WORKERS
CHUNK = 16
N_CHUNKS = B_PER_W // CHUNK
PAIRS = N_CHUNKS // 2

_TAKE_DNUMS = lax.GatherDimensionNumbers(
    offset_dims=(), collapsed_slice_dims=(0,), start_index_map=(0,))


def _lane_permute(v, perm):
    return lax.gather(v, perm[:, None], dimension_numbers=_TAKE_DNUMS,
                      slice_sizes=(1,),
                      mode=lax.GatherScatterMode.PROMISE_IN_BOUNDS)


def _lane_sum(v, perms):
    for perm in perms:
        v = v + _lane_permute(v, perm)
    return v


def _rsqrt_nr(x):
    xi = lax.bitcast_convert_type(x, jnp.int32)
    yi = 0x5F3759DF - lax.shift_right_arithmetic(xi, 1)
    y = lax.bitcast_convert_type(yi, jnp.float32)
    for _ in range(2):
        y = y * (1.5 - 0.5 * x * y * y)
    return y


def _transe_sc(batch_h, batch_t, batch_r, ent_emb, rel_emb):
    mesh = plsc.VectorSubcoreMesh(core_axis_name="c", subcore_axis_name="s")

    @functools.partial(
        pl.kernel,
        mesh=mesh,
        out_type=jax.ShapeDtypeStruct((BATCH,), jnp.float32),
        scratch_types=[
            pltpu.VMEM((B_PER_W,), jnp.int32),
            pltpu.VMEM((B_PER_W,), jnp.int32),
            pltpu.VMEM((B_PER_W,), jnp.int32),
            pltpu.VMEM((2, CHUNK, SUB, DIM), jnp.float32),
            pltpu.VMEM((2, CHUNK, SUB, DIM), jnp.float32),
            pltpu.VMEM((2, CHUNK, SUB, DIM), jnp.float32),
            pltpu.VMEM((B_PER_W,), jnp.float32),
            [pltpu.SemaphoreType.DMA] * 8,
        ],
    )
    def k(bh_hbm, bt_hbm, br_hbm, ent_hbm, rel_hbm, out_hbm,
          idx_h, idx_t, idx_r, h_buf, t_buf, r_buf, out_v, sems):
        wid = lax.axis_index("s") * NUM_CORES + lax.axis_index("c")
        base = wid * B_PER_W

        ent_t = ent_hbm.reshape(ENT_TOT // SUB, SUB, DIM)
        rel_t = rel_hbm.reshape(REL_TOT // SUB, SUB, DIM)

        pltpu.sync_copy(bh_hbm.at[pl.ds(base, B_PER_W)], idx_h)
        pltpu.sync_copy(bt_hbm.at[pl.ds(base, B_PER_W)], idx_t)
        pltpu.sync_copy(br_hbm.at[pl.ds(base, B_PER_W)], idx_r)

        iota16 = lax.iota(jnp.int32, 16)
        perms = [lax.bitwise_xor(iota16, jnp.int32(kk)) for kk in (1, 2, 4, 8)]

        def fire(c, b):
            hvec = idx_h[pl.ds(c * CHUNK, CHUNK)]
            tvec = idx_t[pl.ds(c * CHUNK, CHUNK)]
            rvec = idx_r[pl.ds(c * CHUNK, CHUNK)]
            cps = []
            n = 0
            for vec, tab, buf in ((hvec, ent_t, h_buf),
                                  (tvec, ent_t, t_buf),
                                  (rvec, rel_t, r_buf)):
                for j in range(CHUNK):
                    i = vec[j]
                    ts = lax.shift_right_logical(i, 3)
                    ss = lax.bitwise_and(i, 7)
                    cps.append(pltpu.async_copy(
                        tab.at[ts, ss], buf.at[b, j, ss],
                        sems[4 * b + (n % 4)]))
                    n += 1
            return cps

        def compute(c, b):
            hvec = idx_h[pl.ds(c * CHUNK, CHUNK)]
            tvec = idx_t[pl.ds(c * CHUNK, CHUNK)]
            rvec = idx_r[pl.ds(c * CHUNK, CHUNK)]
            acc = jnp.zeros((16,), jnp.float32)
            for j in range(CHUNK):
                sh = lax.bitwise_and(hvec[j], 7)
                st = lax.bitwise_and(tvec[j], 7)
                sr = lax.bitwise_and(rvec[j], 7)
                hv = [h_buf[b, j, sh, pl.ds(16 * q, 16)] for q in range(4)]
                tv = [t_buf[b, j, st, pl.ds(16 * q, 16)] for q in range(4)]
                rv = [r_buf[b, j, sr, pl.ds(16 * q, 16)] for q in range(4)]

                def inv_norm(vs):
                    ssq = ((vs[0] * vs[0] + vs[1] * vs[1])
                           + (vs[2] * vs[2] + vs[3] * vs[3]))
                    return _rsqrt_nr(jnp.maximum(_lane_sum(ssq, perms), 1e-24))

                ih = inv_norm(hv)
                it = inv_norm(tv)
                ir = inv_norm(rv)

                s = jnp.zeros((16,), jnp.float32)
                for q in range(4):
                    s = s + jnp.abs(hv[q] * ih + rv[q] * ir - tv[q] * it)
                score = _lane_sum(s, perms)
                acc = jnp.where(iota16 == j, score, acc)
            out_v[pl.ds(c * CHUNK, 16)] = acc

        def pair_body(g, _):
            c0 = 2 * g
            cps_a = fire(c0, 0)
            cps_b = fire(c0 + 1, 1)
            for cp in cps_a:
                cp.wait()
            compute(c0, 0)
            for cp in cps_b:
                cp.wait()
            compute(c0 + 1, 1)
            return 0

        lax.fori_loop(0, PAIRS, pair_body, 0)

        pltpu.sync_copy(out_v, out_hbm.at[pl.ds(base, B_PER_W)])

    return k(batch_h, batch_t, batch_r, ent_emb, rel_emb)


def kernel(batch_h, batch_t, batch_r, ent_emb, rel_emb):
    return _transe_sc(batch_h, batch_t, batch_r, ent_emb, rel_emb)
